# baseline (speedup 1.0000x reference)
import functools

import jax
import jax.numpy as jnp
from jax import lax
from jax.experimental import pallas as pl
from jax.experimental.pallas import tpu as pltpu
from jax.experimental.pallas import tpu_sc as plsc

N = 10000
E = 160000
F_IN = 128
H1 = 64
H2 = 32
DP = 128

_NC = 2
_NS = 16
_NW = _NC * _NS
_CHUNK = 128
_CPW = 40
_EP = _NW * _CPW * _CHUNK
_RPT = 624
_RPT_REM = N - _NS * _RPT


def _make_sc_aggregate(d_real):
    mesh = plsc.VectorSubcoreMesh(core_axis_name="c", subcore_axis_name="s")

    @functools.partial(
        pl.kernel,
        mesh=mesh,
        out_type=jax.ShapeDtypeStruct((_NC, N, DP), jnp.float32),
        scratch_types=[
            pltpu.VMEM((_CPW, _CHUNK), jnp.int32),
            pltpu.VMEM((_CPW, _CHUNK), jnp.int32),
            pltpu.VMEM((_CHUNK, DP), jnp.float32),
            pltpu.VMEM((_CHUNK, DP), jnp.float32),
            pltpu.VMEM((_CHUNK * 16,), jnp.float32),
            pltpu.VMEM((_CHUNK * 16,), jnp.float32),
            pltpu.VMEM_SHARED((N, DP), jnp.float32),
            pltpu.SemaphoreType.DMA,
            pltpu.SemaphoreType.DMA,
        ],
    )
    def sc_aggregate(support_hbm, srcs_hbm, dsts_hbm, wx_hbm, zeros_hbm,
                     out_hbm, src_v, dst_v, rows0, rows1, wx0, wx1, acc_sh,
                     gsem, ssem):
        cid = lax.axis_index("c")
        sid = lax.axis_index("s")
        wid = cid * _NS + sid

        row0 = sid * _RPT
        pltpu.sync_copy(zeros_hbm.at[pl.ds(row0, _RPT)],
                        acc_sh.at[pl.ds(row0, _RPT)])

        @pl.when(sid == 0)
        def _():
            pltpu.sync_copy(zeros_hbm.at[pl.ds(_NS * _RPT, _RPT_REM)],
                            acc_sh.at[pl.ds(_NS * _RPT, _RPT_REM)])

        pltpu.sync_copy(srcs_hbm.at[wid], src_v)
        pltpu.sync_copy(dsts_hbm.at[wid], dst_v)
        plsc.subcore_barrier()

        wx_base = wid * (_CPW * _CHUNK * 16)

        def scale(rows_v, wx_v):
            for e in range(_CHUNK):
                wv = wx_v[pl.ds(e * 16, 16)]
                for j in range(d_real // 16):
                    sl = pl.ds(j * 16, 16)
                    rows_v[e, sl] = rows_v[e, sl] * wv

        def pair_body(i, carry):
            c0 = 2 * i
            c1 = c0 + 1
            g0 = pltpu.async_copy(support_hbm.at[src_v.at[c0]], rows0, gsem)
            w0 = pltpu.async_copy(
                wx_hbm.at[pl.ds(wx_base + c0 * (_CHUNK * 16), _CHUNK * 16)],
                wx0, gsem)
            g1 = pltpu.async_copy(support_hbm.at[src_v.at[c1]], rows1, gsem)
            w1 = pltpu.async_copy(
                wx_hbm.at[pl.ds(wx_base + c1 * (_CHUNK * 16), _CHUNK * 16)],
                wx1, gsem)
            g0.wait()
            w0.wait()
            scale(rows0, wx0)
            s0 = pltpu.async_copy(rows0, acc_sh.at[dst_v.at[c0]], ssem,
                                  add=True)
            g1.wait()
            w1.wait()
            scale(rows1, wx1)
            s1 = pltpu.async_copy(rows1, acc_sh.at[dst_v.at[c1]], ssem,
                                  add=True)
            s0.wait()
            s1.wait()
            return carry
        lax.fori_loop(0, _CPW // 2, pair_body, 0)

        plsc.subcore_barrier()
        pltpu.sync_copy(acc_sh.at[pl.ds(row0, _RPT)],
                        out_hbm.at[cid, pl.ds(row0, _RPT)])

        @pl.when(sid == 0)
        def _():
            pltpu.sync_copy(acc_sh.at[pl.ds(_NS * _RPT, _RPT_REM)],
                            out_hbm.at[cid, pl.ds(_NS * _RPT, _RPT_REM)])

    return sc_aggregate


_sc_aggregate_h1 = _make_sc_aggregate(H1)
_sc_aggregate_h2 = _make_sc_aggregate(H2)


def _mm_body(x_ref, w_ref, o_ref):
    o_ref[...] = jnp.dot(x_ref[...], w_ref[...],
                         preferred_element_type=jnp.float32)


def _relu_mm_body(acc_ref, w_ref, o_ref):
    h = jnp.maximum(acc_ref[0] + acc_ref[1], 0.0)
    o_ref[...] = jnp.dot(h, w_ref[...], preferred_element_type=jnp.float32)


def _relu_bn_mm_body(acc_ref, g_ref, b_ref, w_ref, o_ref):
    h = jnp.maximum(acc_ref[0] + acc_ref[1], 0.0)
    mu = jnp.mean(h, axis=0, keepdims=True)
    var = jnp.mean((h - mu) ** 2, axis=0, keepdims=True)
    hn = (h - mu) * lax.rsqrt(var + 1e-5) * g_ref[...] + b_ref[...]
    o_ref[...] = jnp.dot(hn, w_ref[...], preferred_element_type=jnp.float32)


def _relu_bn_body(acc_ref, g_ref, b_ref, o_ref):
    h = jnp.maximum(acc_ref[0] + acc_ref[1], 0.0)
    mu = jnp.mean(h, axis=0, keepdims=True)
    var = jnp.mean((h - mu) ** 2, axis=0, keepdims=True)
    o_ref[...] = (h - mu) * lax.rsqrt(var + 1e-5) * g_ref[...] + b_ref[...]


def _gram_body(a_ref, b_ref, o_ref):
    o_ref[...] = lax.dot_general(a_ref[...], b_ref[...],
                                 (((1,), (1,)), ((), ())),
                                 preferred_element_type=jnp.float32)


_GRAM_BLK = 512


def _pad_cols(a, width=DP):
    return jnp.pad(a, ((0, 0), (0, width - a.shape[1])))


def kernel(x, edge_index, edge_weight, W1, W2, W3,
           bn2_gamma, bn2_beta, bnd_gamma, bnd_beta):
    src = edge_index[0].astype(jnp.int32)
    dst = edge_index[1].astype(jnp.int32)
    w = edge_weight.astype(jnp.float32)

    pad = _EP - E
    pad_idx = (jnp.arange(pad, dtype=jnp.int32) * 13) % N
    srcs = jnp.concatenate([src, pad_idx]).reshape(_NW, _CPW, _CHUNK)
    dsts = jnp.concatenate([dst, pad_idx]).reshape(_NW, _CPW, _CHUNK)
    wp = jnp.concatenate([w, jnp.zeros((pad,), jnp.float32)])
    wx = jnp.broadcast_to(wp[:, None], (_EP, 16)).reshape(_EP * 16)

    zeros = jnp.zeros((N, DP), jnp.float32)
    W1p = _pad_cols(W1)
    W2p = jnp.pad(W2, ((0, DP - H1), (0, DP - H2)))
    W3p = jnp.pad(W3, ((0, DP - H2), (0, DP - H1)))
    g2 = jnp.pad(bn2_gamma, (0, DP - H2)).reshape(1, DP)
    b2 = jnp.pad(bn2_beta, (0, DP - H2)).reshape(1, DP)
    gd = jnp.pad(bnd_gamma, (0, DP - H1)).reshape(1, DP)
    bd = jnp.pad(bnd_beta, (0, DP - H1)).reshape(1, DP)

    support1 = pl.pallas_call(
        _mm_body,
        out_shape=jax.ShapeDtypeStruct((N, DP), jnp.float32),
    )(x, W1p)
    acc1 = _sc_aggregate_h1(support1, srcs, dsts, wx, zeros)

    support2 = pl.pallas_call(
        _relu_mm_body,
        out_shape=jax.ShapeDtypeStruct((N, DP), jnp.float32),
    )(acc1, W2p)
    acc2 = _sc_aggregate_h2(support2, srcs, dsts, wx, zeros)

    support3 = pl.pallas_call(
        _relu_bn_mm_body,
        out_shape=jax.ShapeDtypeStruct((N, DP), jnp.float32),
    )(acc2, g2, b2, W3p)
    acc3 = _sc_aggregate_h1(support3, srcs, dsts, wx, zeros)

    d1 = pl.pallas_call(
        _relu_bn_body,
        out_shape=jax.ShapeDtypeStruct((N, DP), jnp.float32),
    )(acc3, gd, bd)

    out = pl.pallas_call(
        _gram_body,
        grid=(pl.cdiv(N, _GRAM_BLK),),
        in_specs=[pl.BlockSpec((_GRAM_BLK, DP), lambda i: (i, 0)),
                  pl.BlockSpec((N, DP), lambda i: (0, 0))],
        out_specs=pl.BlockSpec((_GRAM_BLK, N), lambda i: (i, 0)),
        out_shape=jax.ShapeDtypeStruct((N, N), jnp.float32),
    )(d1, d1)
    return out


# speedup vs baseline: 1.2945x; 1.2945x over previous
import functools

import jax
import jax.numpy as jnp
from jax import lax
from jax.experimental import pallas as pl
from jax.experimental.pallas import tpu as pltpu
from jax.experimental.pallas import tpu_sc as plsc

N = 10000
E = 160000
F_IN = 128
H1 = 64
H2 = 32
DP = 128

_NC = 2
_NS = 16
_NW = _NC * _NS
_CHUNK = 128
_CPW = 40
_EP = _NW * _CPW * _CHUNK
_RPT = 624
_RPT_REM = N - _NS * _RPT


def _make_sc_aggregate(d_real):
    mesh = plsc.VectorSubcoreMesh(core_axis_name="c", subcore_axis_name="s")

    @functools.partial(
        pl.kernel,
        mesh=mesh,
        out_type=jax.ShapeDtypeStruct((_NC, N, DP), jnp.float32),
        scratch_types=[
            pltpu.VMEM((_CPW, _CHUNK), jnp.int32),
            pltpu.VMEM((_CPW, _CHUNK), jnp.int32),
            pltpu.VMEM((_CPW, _CHUNK), jnp.float32),
            pltpu.VMEM((_CHUNK, DP), jnp.float32),
            pltpu.VMEM((_CHUNK, DP), jnp.float32),
            pltpu.VMEM_SHARED((N, DP), jnp.float32),
            pltpu.SemaphoreType.DMA,
            pltpu.SemaphoreType.DMA,
        ],
    )
    def sc_aggregate(support_hbm, srcs_hbm, dsts_hbm, ws_hbm, zeros_hbm,
                     out_hbm, src_v, dst_v, w_v, rows0, rows1, acc_sh,
                     gsem, ssem):
        cid = lax.axis_index("c")
        sid = lax.axis_index("s")
        wid = cid * _NS + sid

        row0 = sid * _RPT
        pltpu.sync_copy(zeros_hbm.at[pl.ds(row0, _RPT)],
                        acc_sh.at[pl.ds(row0, _RPT)])

        @pl.when(sid == 0)
        def _():
            pltpu.sync_copy(zeros_hbm.at[pl.ds(_NS * _RPT, _RPT_REM)],
                            acc_sh.at[pl.ds(_NS * _RPT, _RPT_REM)])

        pltpu.sync_copy(srcs_hbm.at[wid], src_v)
        pltpu.sync_copy(dsts_hbm.at[wid], dst_v)
        pltpu.sync_copy(ws_hbm.at[wid], w_v)
        plsc.subcore_barrier()

        def scale(rows_v, c):
            for g in range(_CHUNK // 16):
                w16 = w_v[c, pl.ds(g * 16, 16)]
                for lane in range(16):
                    ws = w16[lane]
                    e = g * 16 + lane
                    for j in range(d_real // 16):
                        sl = pl.ds(j * 16, 16)
                        rows_v[e, sl] = rows_v[e, sl] * ws

        def pair_body(i, carry):
            c0 = 2 * i
            c1 = c0 + 1
            g0 = pltpu.async_copy(support_hbm.at[src_v.at[c0]], rows0, gsem)
            g1 = pltpu.async_copy(support_hbm.at[src_v.at[c1]], rows1, gsem)
            g0.wait()
            scale(rows0, c0)
            s0 = pltpu.async_copy(rows0, acc_sh.at[dst_v.at[c0]], ssem,
                                  add=True)
            g1.wait()
            scale(rows1, c1)
            s1 = pltpu.async_copy(rows1, acc_sh.at[dst_v.at[c1]], ssem,
                                  add=True)
            s0.wait()
            s1.wait()
            return carry
        lax.fori_loop(0, _CPW // 2, pair_body, 0)

        plsc.subcore_barrier()
        pltpu.sync_copy(acc_sh.at[pl.ds(row0, _RPT)],
                        out_hbm.at[cid, pl.ds(row0, _RPT)])

        @pl.when(sid == 0)
        def _():
            pltpu.sync_copy(acc_sh.at[pl.ds(_NS * _RPT, _RPT_REM)],
                            out_hbm.at[cid, pl.ds(_NS * _RPT, _RPT_REM)])

    return sc_aggregate


_sc_aggregate_h1 = _make_sc_aggregate(H1)
_sc_aggregate_h2 = _make_sc_aggregate(H2)


def _mm_body(x_ref, w_ref, o_ref):
    o_ref[...] = jnp.dot(x_ref[...], w_ref[...],
                         preferred_element_type=jnp.float32)


def _relu_mm_body(acc_ref, w_ref, o_ref):
    h = jnp.maximum(acc_ref[0] + acc_ref[1], 0.0)
    o_ref[...] = jnp.dot(h, w_ref[...], preferred_element_type=jnp.float32)


def _relu_bn_mm_body(acc_ref, g_ref, b_ref, w_ref, o_ref):
    h = jnp.maximum(acc_ref[0] + acc_ref[1], 0.0)
    mu = jnp.mean(h, axis=0, keepdims=True)
    var = jnp.mean((h - mu) ** 2, axis=0, keepdims=True)
    hn = (h - mu) * lax.rsqrt(var + 1e-5) * g_ref[...] + b_ref[...]
    o_ref[...] = jnp.dot(hn, w_ref[...], preferred_element_type=jnp.float32)


def _relu_bn_body(acc_ref, g_ref, b_ref, o_ref):
    h = jnp.maximum(acc_ref[0] + acc_ref[1], 0.0)
    mu = jnp.mean(h, axis=0, keepdims=True)
    var = jnp.mean((h - mu) ** 2, axis=0, keepdims=True)
    o_ref[...] = (h - mu) * lax.rsqrt(var + 1e-5) * g_ref[...] + b_ref[...]


def _gram_body(a_ref, b_ref, o_ref):
    o_ref[...] = lax.dot_general(a_ref[...], b_ref[...],
                                 (((1,), (1,)), ((), ())),
                                 preferred_element_type=jnp.float32)


_GRAM_BLK = 512


def _pad_cols(a, width=DP):
    return jnp.pad(a, ((0, 0), (0, width - a.shape[1])))


def kernel(x, edge_index, edge_weight, W1, W2, W3,
           bn2_gamma, bn2_beta, bnd_gamma, bnd_beta):
    src = edge_index[0].astype(jnp.int32)
    dst = edge_index[1].astype(jnp.int32)
    w = edge_weight.astype(jnp.float32)

    pad = _EP - E
    pad_idx = (jnp.arange(pad, dtype=jnp.int32) * 13) % N
    srcs = jnp.concatenate([src, pad_idx]).reshape(_NW, _CPW, _CHUNK)
    dsts = jnp.concatenate([dst, pad_idx]).reshape(_NW, _CPW, _CHUNK)
    ws = jnp.concatenate([w, jnp.zeros((pad,), jnp.float32)]
                         ).reshape(_NW, _CPW, _CHUNK)

    zeros = jnp.zeros((N, DP), jnp.float32)
    W1p = _pad_cols(W1)
    W2p = jnp.pad(W2, ((0, DP - H1), (0, DP - H2)))
    W3p = jnp.pad(W3, ((0, DP - H2), (0, DP - H1)))
    g2 = jnp.pad(bn2_gamma, (0, DP - H2)).reshape(1, DP)
    b2 = jnp.pad(bn2_beta, (0, DP - H2)).reshape(1, DP)
    gd = jnp.pad(bnd_gamma, (0, DP - H1)).reshape(1, DP)
    bd = jnp.pad(bnd_beta, (0, DP - H1)).reshape(1, DP)

    support1 = pl.pallas_call(
        _mm_body,
        out_shape=jax.ShapeDtypeStruct((N, DP), jnp.float32),
    )(x, W1p)
    acc1 = _sc_aggregate_h1(support1, srcs, dsts, ws, zeros)

    support2 = pl.pallas_call(
        _relu_mm_body,
        out_shape=jax.ShapeDtypeStruct((N, DP), jnp.float32),
    )(acc1, W2p)
    acc2 = _sc_aggregate_h2(support2, srcs, dsts, ws, zeros)

    support3 = pl.pallas_call(
        _relu_bn_mm_body,
        out_shape=jax.ShapeDtypeStruct((N, DP), jnp.float32),
    )(acc2, g2, b2, W3p)
    acc3 = _sc_aggregate_h1(support3, srcs, dsts, ws, zeros)

    d1 = pl.pallas_call(
        _relu_bn_body,
        out_shape=jax.ShapeDtypeStruct((N, DP), jnp.float32),
    )(acc3, gd, bd)

    out = pl.pallas_call(
        _gram_body,
        grid=(pl.cdiv(N, _GRAM_BLK),),
        in_specs=[pl.BlockSpec((_GRAM_BLK, DP), lambda i: (i, 0)),
                  pl.BlockSpec((N, DP), lambda i: (0, 0))],
        out_specs=pl.BlockSpec((_GRAM_BLK, N), lambda i: (i, 0)),
        out_shape=jax.ShapeDtypeStruct((N, N), jnp.float32),
    )(d1, d1)
    return out


# speedup vs baseline: 1.5065x; 1.1638x over previous
import functools

import jax
import jax.numpy as jnp
from jax import lax
from jax.experimental import pallas as pl
from jax.experimental.pallas import tpu as pltpu
from jax.experimental.pallas import tpu_sc as plsc

N = 10000
E = 160000
F_IN = 128
H1 = 64
H2 = 32

_NC = 2
_NS = 16
_NW = _NC * _NS
_CHUNK = 128
_CPW = 40
_EP = _NW * _CPW * _CHUNK
_RPT = 624
_RPT_REM = N - _NS * _RPT


def _make_sc_aggregate(D):
    mesh = plsc.VectorSubcoreMesh(core_axis_name="c", subcore_axis_name="s")

    @functools.partial(
        pl.kernel,
        mesh=mesh,
        out_type=jax.ShapeDtypeStruct((_NC, N, D), jnp.float32),
        compiler_params=pltpu.CompilerParams(use_tc_tiling_on_sc=False),
        scratch_types=[
            pltpu.VMEM((_CPW, _CHUNK), jnp.int32),
            pltpu.VMEM((_CPW, _CHUNK), jnp.int32),
            pltpu.VMEM((_CPW, _CHUNK), jnp.float32),
            pltpu.VMEM((_CHUNK, D), jnp.float32),
            pltpu.VMEM((_CHUNK, D), jnp.float32),
            pltpu.VMEM_SHARED((N, D), jnp.float32),
            pltpu.SemaphoreType.DMA,
            pltpu.SemaphoreType.DMA,
        ],
    )
    def sc_aggregate(support_hbm, srcs_hbm, dsts_hbm, ws_hbm, zeros_hbm,
                     out_hbm, src_v, dst_v, w_v, rows0, rows1, acc_sh,
                     gsem, ssem):
        cid = lax.axis_index("c")
        sid = lax.axis_index("s")
        wid = cid * _NS + sid

        row0 = sid * _RPT
        pltpu.sync_copy(zeros_hbm.at[pl.ds(row0, _RPT)],
                        acc_sh.at[pl.ds(row0, _RPT)])

        @pl.when(sid == 0)
        def _():
            pltpu.sync_copy(zeros_hbm.at[pl.ds(_NS * _RPT, _RPT_REM)],
                            acc_sh.at[pl.ds(_NS * _RPT, _RPT_REM)])

        pltpu.sync_copy(srcs_hbm.at[wid], src_v)
        pltpu.sync_copy(dsts_hbm.at[wid], dst_v)
        pltpu.sync_copy(ws_hbm.at[wid], w_v)
        plsc.subcore_barrier()

        def scale(rows_v, c):
            for g in range(_CHUNK // 16):
                w16 = w_v[c, pl.ds(g * 16, 16)]
                for lane in range(16):
                    ws = w16[lane]
                    e = g * 16 + lane
                    for j in range(D // 16):
                        sl = pl.ds(j * 16, 16)
                        rows_v[e, sl] = rows_v[e, sl] * ws

        def pair_body(i, carry):
            c0 = 2 * i
            c1 = c0 + 1
            g0 = pltpu.async_copy(support_hbm.at[src_v.at[c0]], rows0, gsem)
            g1 = pltpu.async_copy(support_hbm.at[src_v.at[c1]], rows1, gsem)
            g0.wait()
            scale(rows0, c0)
            s0 = pltpu.async_copy(rows0, acc_sh.at[dst_v.at[c0]], ssem,
                                  add=True)
            g1.wait()
            scale(rows1, c1)
            s1 = pltpu.async_copy(rows1, acc_sh.at[dst_v.at[c1]], ssem,
                                  add=True)
            s0.wait()
            s1.wait()
            return carry
        lax.fori_loop(0, _CPW // 2, pair_body, 0)

        plsc.subcore_barrier()
        pltpu.sync_copy(acc_sh.at[pl.ds(row0, _RPT)],
                        out_hbm.at[cid, pl.ds(row0, _RPT)])

        @pl.when(sid == 0)
        def _():
            pltpu.sync_copy(acc_sh.at[pl.ds(_NS * _RPT, _RPT_REM)],
                            out_hbm.at[cid, pl.ds(_NS * _RPT, _RPT_REM)])

    return sc_aggregate


_sc_aggregate_h1 = _make_sc_aggregate(H1)
_sc_aggregate_h2 = _make_sc_aggregate(H2)


def _mm_body(x_ref, w_ref, o_ref):
    o_ref[...] = jnp.dot(x_ref[...], w_ref[...],
                         preferred_element_type=jnp.float32)


def _relu_mm_body(acc_ref, w_ref, o_ref):
    h = jnp.maximum(acc_ref[0] + acc_ref[1], 0.0)
    o_ref[...] = jnp.dot(h, w_ref[...], preferred_element_type=jnp.float32)


def _relu_bn_mm_body(acc_ref, g_ref, b_ref, w_ref, o_ref):
    h = jnp.maximum(acc_ref[0] + acc_ref[1], 0.0)
    mu = jnp.mean(h, axis=0, keepdims=True)
    var = jnp.mean((h - mu) ** 2, axis=0, keepdims=True)
    hn = (h - mu) * lax.rsqrt(var + 1e-5) * g_ref[...] + b_ref[...]
    o_ref[...] = jnp.dot(hn, w_ref[...], preferred_element_type=jnp.float32)


def _relu_bn_body(acc_ref, g_ref, b_ref, o_ref):
    h = jnp.maximum(acc_ref[0] + acc_ref[1], 0.0)
    mu = jnp.mean(h, axis=0, keepdims=True)
    var = jnp.mean((h - mu) ** 2, axis=0, keepdims=True)
    o_ref[...] = (h - mu) * lax.rsqrt(var + 1e-5) * g_ref[...] + b_ref[...]


def _gram_body(a_ref, b_ref, o_ref):
    o_ref[...] = lax.dot_general(a_ref[...], b_ref[...],
                                 (((1,), (1,)), ((), ())),
                                 preferred_element_type=jnp.float32)


_GRAM_BLK = 512


def kernel(x, edge_index, edge_weight, W1, W2, W3,
           bn2_gamma, bn2_beta, bnd_gamma, bnd_beta):
    src = edge_index[0].astype(jnp.int32)
    dst = edge_index[1].astype(jnp.int32)
    w = edge_weight.astype(jnp.float32)

    pad = _EP - E
    pad_idx = (jnp.arange(pad, dtype=jnp.int32) * 13) % N
    srcs = jnp.concatenate([src, pad_idx]).reshape(_NW, _CPW, _CHUNK)
    dsts = jnp.concatenate([dst, pad_idx]).reshape(_NW, _CPW, _CHUNK)
    ws = jnp.concatenate([w, jnp.zeros((pad,), jnp.float32)]
                         ).reshape(_NW, _CPW, _CHUNK)

    zeros64 = jnp.zeros((N, H1), jnp.float32)
    zeros32 = jnp.zeros((N, H2), jnp.float32)
    g2 = bn2_gamma.reshape(1, H2)
    b2 = bn2_beta.reshape(1, H2)
    gd = bnd_gamma.reshape(1, H1)
    bd = bnd_beta.reshape(1, H1)

    support1 = pl.pallas_call(
        _mm_body,
        out_shape=jax.ShapeDtypeStruct((N, H1), jnp.float32),
    )(x, W1)
    acc1 = _sc_aggregate_h1(support1, srcs, dsts, ws, zeros64)

    support2 = pl.pallas_call(
        _relu_mm_body,
        out_shape=jax.ShapeDtypeStruct((N, H2), jnp.float32),
    )(acc1, W2)
    acc2 = _sc_aggregate_h2(support2, srcs, dsts, ws, zeros32)

    support3 = pl.pallas_call(
        _relu_bn_mm_body,
        out_shape=jax.ShapeDtypeStruct((N, H1), jnp.float32),
    )(acc2, g2, b2, W3)
    acc3 = _sc_aggregate_h1(support3, srcs, dsts, ws, zeros64)

    d1 = pl.pallas_call(
        _relu_bn_body,
        out_shape=jax.ShapeDtypeStruct((N, H1), jnp.float32),
    )(acc3, gd, bd)

    out = pl.pallas_call(
        _gram_body,
        grid=(pl.cdiv(N, _GRAM_BLK),),
        in_specs=[pl.BlockSpec((_GRAM_BLK, H1), lambda i: (i, 0)),
                  pl.BlockSpec((N, H1), lambda i: (0, 0))],
        out_specs=pl.BlockSpec((_GRAM_BLK, N), lambda i: (i, 0)),
        out_shape=jax.ShapeDtypeStruct((N, N), jnp.float32),
    )(d1, d1)
    return out


# speedup vs baseline: 1.5996x; 1.0618x over previous
import functools

import jax
import jax.numpy as jnp
from jax import lax
from jax.experimental import pallas as pl
from jax.experimental.pallas import tpu as pltpu
from jax.experimental.pallas import tpu_sc as plsc

N = 10000
E = 160000
F_IN = 128
H1 = 64
H2 = 32

_NC = 2
_NS = 16
_NW = _NC * _NS
_CHUNK = 128
_CPW = 40
_EP = _NW * _CPW * _CHUNK
_NB = 8
_LOOKAHEAD = 4
_RPT = 624
_RPT_REM = N - _NS * _RPT


def _make_sc_aggregate(D):
    mesh = plsc.VectorSubcoreMesh(core_axis_name="c", subcore_axis_name="s")

    @functools.partial(
        pl.kernel,
        mesh=mesh,
        out_type=jax.ShapeDtypeStruct((_NC, N, D), jnp.float32),
        compiler_params=pltpu.CompilerParams(use_tc_tiling_on_sc=False),
        scratch_types=[
            pltpu.VMEM((_CPW, _CHUNK), jnp.int32),
            pltpu.VMEM((_CPW, _CHUNK), jnp.int32),
            pltpu.VMEM((_CPW, _CHUNK), jnp.float32),
            pltpu.VMEM((_NB, _CHUNK, D), jnp.float32),
            pltpu.VMEM_SHARED((N, D), jnp.float32),
        ] + [pltpu.SemaphoreType.DMA] * _NB,
    )
    def sc_aggregate(support_hbm, srcs_hbm, dsts_hbm, ws_hbm, zeros_hbm,
                     out_hbm, src_v, dst_v, w_v, rows_ring, acc_sh, *sems):
        cid = lax.axis_index("c")
        sid = lax.axis_index("s")
        wid = cid * _NS + sid

        row0 = sid * _RPT
        pltpu.sync_copy(zeros_hbm.at[pl.ds(row0, _RPT)],
                        acc_sh.at[pl.ds(row0, _RPT)])

        @pl.when(sid == 0)
        def _():
            pltpu.sync_copy(zeros_hbm.at[pl.ds(_NS * _RPT, _RPT_REM)],
                            acc_sh.at[pl.ds(_NS * _RPT, _RPT_REM)])

        pltpu.sync_copy(srcs_hbm.at[wid], src_v)
        pltpu.sync_copy(dsts_hbm.at[wid], dst_v)
        pltpu.sync_copy(ws_hbm.at[wid], w_v)
        plsc.subcore_barrier()

        def scale(b, c):
            def grp(g, carry):
                w16 = w_v[c, pl.ds(g * 16, 16)]
                for lane in range(16):
                    ws = w16[lane]
                    e = g * 16 + lane
                    for j in range(D // 16):
                        sl = pl.ds(j * 16, 16)
                        rows_ring[b, e, sl] = rows_ring[b, e, sl] * ws
                return carry
            lax.fori_loop(0, _CHUNK // 16, grp, 0)

        def drain(b, c):
            pltpu.make_async_copy(support_hbm.at[pl.ds(0, _CHUNK)],
                                  rows_ring.at[b], sems[b]).wait()

        for b in range(_LOOKAHEAD):
            pltpu.async_copy(support_hbm.at[src_v.at[b]], rows_ring.at[b],
                             sems[b])

        def ring_body(i, carry):
            for b in range(_NB):
                c = i * _NB + b
                drain(b, c)
                scale(b, c)
                pltpu.async_copy(rows_ring.at[b], acc_sh.at[dst_v.at[c]],
                                 sems[b], add=True)
                cn = c + _LOOKAHEAD
                bn = (b + _LOOKAHEAD) % _NB

                @pl.when(cn < _CPW)
                def _():
                    @pl.when(cn >= _NB)
                    def _():
                        drain(bn, cn)
                    pltpu.async_copy(support_hbm.at[src_v.at[cn]],
                                     rows_ring.at[bn], sems[bn])
            return carry
        lax.fori_loop(0, _CPW // _NB, ring_body, 0)

        for b in range(_NB):
            drain(b, 0)

        plsc.subcore_barrier()
        pltpu.sync_copy(acc_sh.at[pl.ds(row0, _RPT)],
                        out_hbm.at[cid, pl.ds(row0, _RPT)])

        @pl.when(sid == 0)
        def _():
            pltpu.sync_copy(acc_sh.at[pl.ds(_NS * _RPT, _RPT_REM)],
                            out_hbm.at[cid, pl.ds(_NS * _RPT, _RPT_REM)])

    return sc_aggregate


_sc_aggregate_h1 = _make_sc_aggregate(H1)
_sc_aggregate_h2 = _make_sc_aggregate(H2)


def _mm_body(x_ref, w_ref, o_ref):
    o_ref[...] = jnp.dot(x_ref[...], w_ref[...],
                         preferred_element_type=jnp.float32)


def _relu_mm_body(acc_ref, w_ref, o_ref):
    h = jnp.maximum(acc_ref[0] + acc_ref[1], 0.0)
    o_ref[...] = jnp.dot(h, w_ref[...], preferred_element_type=jnp.float32)


def _relu_bn_mm_body(acc_ref, g_ref, b_ref, w_ref, o_ref):
    h = jnp.maximum(acc_ref[0] + acc_ref[1], 0.0)
    mu = jnp.mean(h, axis=0, keepdims=True)
    var = jnp.mean((h - mu) ** 2, axis=0, keepdims=True)
    hn = (h - mu) * lax.rsqrt(var + 1e-5) * g_ref[...] + b_ref[...]
    o_ref[...] = jnp.dot(hn, w_ref[...], preferred_element_type=jnp.float32)


def _relu_bn_body(acc_ref, g_ref, b_ref, o_ref):
    h = jnp.maximum(acc_ref[0] + acc_ref[1], 0.0)
    mu = jnp.mean(h, axis=0, keepdims=True)
    var = jnp.mean((h - mu) ** 2, axis=0, keepdims=True)
    o_ref[...] = (h - mu) * lax.rsqrt(var + 1e-5) * g_ref[...] + b_ref[...]


def _gram_body(a_ref, b_ref, o_ref):
    o_ref[...] = lax.dot_general(a_ref[...], b_ref[...],
                                 (((1,), (1,)), ((), ())),
                                 preferred_element_type=jnp.float32)


_GRAM_BLK = 512


def kernel(x, edge_index, edge_weight, W1, W2, W3,
           bn2_gamma, bn2_beta, bnd_gamma, bnd_beta):
    src = edge_index[0].astype(jnp.int32)
    dst = edge_index[1].astype(jnp.int32)
    w = edge_weight.astype(jnp.float32)

    pad = _EP - E
    pad_idx = (jnp.arange(pad, dtype=jnp.int32) * 13) % N
    srcs = jnp.concatenate([src, pad_idx]).reshape(_NW, _CPW, _CHUNK)
    dsts = jnp.concatenate([dst, pad_idx]).reshape(_NW, _CPW, _CHUNK)
    ws = jnp.concatenate([w, jnp.zeros((pad,), jnp.float32)]
                         ).reshape(_NW, _CPW, _CHUNK)

    zeros64 = jnp.zeros((N, H1), jnp.float32)
    zeros32 = jnp.zeros((N, H2), jnp.float32)
    g2 = bn2_gamma.reshape(1, H2)
    b2 = bn2_beta.reshape(1, H2)
    gd = bnd_gamma.reshape(1, H1)
    bd = bnd_beta.reshape(1, H1)

    support1 = pl.pallas_call(
        _mm_body,
        out_shape=jax.ShapeDtypeStruct((N, H1), jnp.float32),
    )(x, W1)
    acc1 = _sc_aggregate_h1(support1, srcs, dsts, ws, zeros64)

    support2 = pl.pallas_call(
        _relu_mm_body,
        out_shape=jax.ShapeDtypeStruct((N, H2), jnp.float32),
    )(acc1, W2)
    acc2 = _sc_aggregate_h2(support2, srcs, dsts, ws, zeros32)

    support3 = pl.pallas_call(
        _relu_bn_mm_body,
        out_shape=jax.ShapeDtypeStruct((N, H1), jnp.float32),
    )(acc2, g2, b2, W3)
    acc3 = _sc_aggregate_h1(support3, srcs, dsts, ws, zeros64)

    d1 = pl.pallas_call(
        _relu_bn_body,
        out_shape=jax.ShapeDtypeStruct((N, H1), jnp.float32),
    )(acc3, gd, bd)

    out = pl.pallas_call(
        _gram_body,
        grid=(pl.cdiv(N, _GRAM_BLK),),
        in_specs=[pl.BlockSpec((_GRAM_BLK, H1), lambda i: (i, 0)),
                  pl.BlockSpec((N, H1), lambda i: (0, 0))],
        out_specs=pl.BlockSpec((_GRAM_BLK, N), lambda i: (i, 0)),
        out_shape=jax.ShapeDtypeStruct((N, N), jnp.float32),
    )(d1, d1)
    return out
